# baseline (speedup 1.0000x reference)
import functools

import jax
import jax.numpy as jnp
from jax import lax
from jax.experimental import pallas as pl
from jax.experimental.pallas import tpu as pltpu
from jax.experimental.pallas import tpu_sc as plsc

VOCAB = 1000000
EMBED_DIM = 64

try:
    _INFO = plsc.get_sparse_core_info()
    _NC = _INFO.num_cores
    _NS = _INFO.num_subcores
except Exception:
    _NC, _NS = 2, 16
_NW = _NC * _NS

_LANES = 128


def _make_gather(n_units: int, seq: int, batch: int):
    mesh = plsc.VectorSubcoreMesh(core_axis_name="c", subcore_axis_name="s")
    u_per_w = n_units // _NW

    @functools.partial(
        pl.kernel,
        out_type=jax.ShapeDtypeStruct((seq, EMBED_DIM, batch), jnp.float32),
        mesh=mesh,
        scratch_types=[
            pltpu.VMEM((u_per_w, _LANES), jnp.int32),
            pltpu.VMEM((u_per_w, _LANES), jnp.int32),
            pltpu.VMEM((2, _LANES, 2 * EMBED_DIM), jnp.float32),
            pltpu.VMEM((2, EMBED_DIM, _LANES), jnp.float32),
            pltpu.SemaphoreType.DMA,
            pltpu.SemaphoreType.DMA,
            pltpu.SemaphoreType.DMA,
            pltpu.SemaphoreType.DMA,
        ],
        compiler_params=pltpu.CompilerParams(use_tc_tiling_on_sc=True,
                                             needs_layout_passes=False),
    )
    def gather_kernel(rows_hbm, half_hbm, table_hbm, out_hbm, rows_v, half_v,
                      gbuf, tbuf, sem_g0, sem_g1, sem_w0, sem_w1):
        wid = lax.axis_index("s") * _NC + lax.axis_index("c")
        ubase = wid * u_per_w
        pltpu.sync_copy(rows_hbm.at[pl.ds(ubase, u_per_w)], rows_v)
        pltpu.sync_copy(half_hbm.at[pl.ds(ubase, u_per_w)], half_v)

        def g_copy(u, b, sem):
            return pltpu.make_async_copy(table_hbm.at[rows_v.at[u]],
                                         gbuf.at[b], sem)

        def w_copy(u, b, sem):
            gu = ubase + u
            s = gu // (batch // _LANES)
            bc = gu % (batch // _LANES)
            return pltpu.make_async_copy(
                tbuf.at[b], out_hbm.at[s, :, pl.ds(bc * _LANES, _LANES)], sem)

        def extract(u, b):
            cvs = [half_v[u, pl.ds(16 * g, 16)] for g in range(8)]
            rvs = [lax.iota(jnp.int32, 16) + 16 * g for g in range(8)]

            def jbody(j, carry):
                for g in range(8):
                    vals = plsc.load_gather(gbuf.at[b], [rvs[g], cvs[g] + j])
                    tbuf[b, j, pl.ds(16 * g, 16)] = vals
                return carry

            lax.fori_loop(0, EMBED_DIM, jbody, 0, unroll=2)

        g_copy(0, 0, sem_g0).start()

        def unit_body(u, carry):
            b = lax.rem(u, 2)

            @pl.when(u + 1 < u_per_w)
            def _():
                @pl.when(b == 0)
                def _():
                    g_copy(u + 1, 1, sem_g1).start()

                @pl.when(b == 1)
                def _():
                    g_copy(u + 1, 0, sem_g0).start()

            @pl.when(b == 0)
            def _():
                g_copy(u, 0, sem_g0).wait()

                @pl.when(u >= 2)
                def _():
                    w_copy(u - 2, 0, sem_w0).wait()

                extract(u, 0)
                w_copy(u, 0, sem_w0).start()

            @pl.when(b == 1)
            def _():
                g_copy(u, 1, sem_g1).wait()

                @pl.when(u >= 2)
                def _():
                    w_copy(u - 2, 1, sem_w1).wait()

                extract(u, 1)
                w_copy(u, 1, sem_w1).start()

            return carry

        lax.fori_loop(0, u_per_w, unit_body, 0)
        w_copy(u_per_w - 2, 0, sem_w0).wait()
        w_copy(u_per_w - 1, 1, sem_w1).wait()

    return gather_kernel


def kernel(tokens, weight):
    b, s = tokens.shape
    assert b % _LANES == 0
    n_units = s * (b // _LANES)
    tok = tokens.astype(jnp.int32).T.reshape(n_units, _LANES)
    rows = tok >> 1
    half = (tok & 1) * EMBED_DIM
    table = weight.reshape(VOCAB // 2, 2 * EMBED_DIM)
    out = _make_gather(n_units, s, b)(rows, half, table)
    return jnp.transpose(out, (2, 0, 1))


# speedup vs baseline: 1.4568x; 1.4568x over previous
import functools

import jax
import jax.numpy as jnp
from jax import lax
from jax.experimental import pallas as pl
from jax.experimental.pallas import tpu as pltpu
from jax.experimental.pallas import tpu_sc as plsc

VOCAB = 1000000
EMBED_DIM = 64

try:
    _INFO = plsc.get_sparse_core_info()
    _NC = _INFO.num_cores
    _NS = _INFO.num_subcores
except Exception:
    _NC, _NS = 2, 16
_NW = _NC * _NS

_LANES = 128


def _make_gather(n_units: int, seq: int, batch: int):
    mesh = plsc.VectorSubcoreMesh(core_axis_name="c", subcore_axis_name="s")
    u_per_w = n_units // _NW

    @functools.partial(
        pl.kernel,
        out_type=jax.ShapeDtypeStruct((seq, EMBED_DIM, batch), jnp.float32),
        mesh=mesh,
        scratch_types=[
            pltpu.VMEM((u_per_w, _LANES), jnp.int32),
            pltpu.VMEM((u_per_w, _LANES), jnp.int32),
            pltpu.VMEM((2, _LANES, 2 * EMBED_DIM), jnp.float32),
            pltpu.VMEM((2, EMBED_DIM, _LANES), jnp.float32),
            pltpu.SemaphoreType.DMA((2,)),
            pltpu.SemaphoreType.DMA((2,)),
        ],
        compiler_params=pltpu.CompilerParams(use_tc_tiling_on_sc=True,
                                             needs_layout_passes=False),
    )
    def gather_kernel(rows_hbm, half_hbm, table_hbm, out_hbm, rows_v, half_v,
                      gbuf, tbuf, sem_g, sem_w):
        wid = lax.axis_index("s") * _NC + lax.axis_index("c")
        ubase = wid * u_per_w
        pltpu.sync_copy(rows_hbm.at[pl.ds(ubase, u_per_w)], rows_v)
        pltpu.sync_copy(half_hbm.at[pl.ds(ubase, u_per_w)], half_v)

        rvs = [lax.iota(jnp.int32, 16) + 16 * g for g in range(8)]

        def g_copy(u, b):
            return pltpu.make_async_copy(table_hbm.at[rows_v.at[u]],
                                         gbuf.at[b], sem_g.at[b])

        def w_copy(u, b):
            gu = ubase + u
            s = gu // (batch // _LANES)
            bc = gu % (batch // _LANES)
            return pltpu.make_async_copy(
                tbuf.at[b], out_hbm.at[s, :, pl.ds(bc * _LANES, _LANES)],
                sem_w.at[b])

        g_copy(0, 0).start()

        def unit_body(u, carry):
            b = lax.rem(u, 2)

            @pl.when(u + 1 < u_per_w)
            def _():
                g_copy(u + 1, 1 - b).start()

            g_copy(u, b).wait()

            @pl.when(u >= 2)
            def _():
                w_copy(u - 2, b).wait()

            cvs = [half_v[u, pl.ds(16 * g, 16)] for g in range(8)]

            @plsc.parallel_loop(0, EMBED_DIM, unroll=4)
            def _(j):
                for g in range(8):
                    vals = plsc.load_gather(gbuf.at[b], [rvs[g], cvs[g] + j])
                    tbuf[b, j, pl.ds(16 * g, 16)] = vals

            w_copy(u, b).start()
            return carry

        lax.fori_loop(0, u_per_w, unit_body, 0)
        w_copy(u_per_w - 2, 0).wait()
        w_copy(u_per_w - 1, 1).wait()

    return gather_kernel


def kernel(tokens, weight):
    b, s = tokens.shape
    assert b % _LANES == 0
    n_units = s * (b // _LANES)
    tok = tokens.astype(jnp.int32).T.reshape(n_units, _LANES)
    rows = tok >> 1
    half = (tok & 1) * EMBED_DIM
    table = weight.reshape(VOCAB // 2, 2 * EMBED_DIM)
    out = _make_gather(n_units, s, b)(rows, half, table)
    return jnp.transpose(out, (2, 0, 1))


# speedup vs baseline: 1.4577x; 1.0006x over previous
import functools

import jax
import jax.numpy as jnp
from jax import lax
from jax.experimental import pallas as pl
from jax.experimental.pallas import tpu as pltpu
from jax.experimental.pallas import tpu_sc as plsc

VOCAB = 1000000
EMBED_DIM = 64

try:
    _INFO = plsc.get_sparse_core_info()
    _NC = _INFO.num_cores
    _NS = _INFO.num_subcores
except Exception:
    _NC, _NS = 2, 16
_NW = _NC * _NS

_LANES = 128


def _make_gather(n_units: int, seq: int, batch: int):
    mesh = plsc.VectorSubcoreMesh(core_axis_name="c", subcore_axis_name="s")
    u_per_w = n_units // _NW

    @functools.partial(
        pl.kernel,
        out_type=jax.ShapeDtypeStruct((seq, EMBED_DIM, batch), jnp.float32),
        mesh=mesh,
        scratch_types=[
            pltpu.VMEM((u_per_w, _LANES), jnp.int32),
            pltpu.VMEM((u_per_w, _LANES), jnp.int32),
            pltpu.VMEM((3, _LANES, 2 * EMBED_DIM), jnp.float32),
            pltpu.VMEM((2, EMBED_DIM, _LANES), jnp.float32),
            pltpu.SemaphoreType.DMA((2,)),
            pltpu.SemaphoreType.DMA((2,)),
        ],
        compiler_params=pltpu.CompilerParams(use_tc_tiling_on_sc=True,
                                             needs_layout_passes=False),
    )
    def gather_kernel(rows_hbm, half_hbm, table_hbm, out_hbm, rows_v, half_v,
                      gbuf, tbuf, sem_g, sem_w):
        wid = lax.axis_index("s") * _NC + lax.axis_index("c")
        ubase = wid * u_per_w
        pltpu.sync_copy(rows_hbm.at[pl.ds(ubase, u_per_w)], rows_v)
        pltpu.sync_copy(half_hbm.at[pl.ds(ubase, u_per_w)], half_v)

        rvs = [lax.iota(jnp.int32, 16) + 16 * g for g in range(8)]

        def g_copy(u, b):
            return pltpu.make_async_copy(table_hbm.at[rows_v.at[u]],
                                         gbuf.at[b], sem_g.at[b])

        def w_copy(u, b):
            gu = ubase + u
            s = gu // (batch // _LANES)
            bc = gu % (batch // _LANES)
            return pltpu.make_async_copy(
                tbuf.at[b], out_hbm.at[s, :, pl.ds(bc * _LANES, _LANES)],
                sem_w.at[b])

        g_copy(0, 0).start()
        g_copy(1, 1).start()

        def unit_body(u, carry):
            bg = lax.rem(u, 3)
            bt = lax.rem(u, 2)

            @pl.when(u + 2 < u_per_w)
            def _():
                g_copy(u + 2, lax.rem(u + 2, 3)).start()

            g_copy(u, bg).wait()

            @pl.when(u >= 2)
            def _():
                w_copy(u - 2, bt).wait()

            cvs = [half_v[u, pl.ds(16 * g, 16)] for g in range(8)]

            @plsc.parallel_loop(0, EMBED_DIM, unroll=8)
            def _(j):
                for g in range(8):
                    vals = plsc.load_gather(gbuf.at[bg], [rvs[g], cvs[g] + j])
                    tbuf[bt, j, pl.ds(16 * g, 16)] = vals

            w_copy(u, bt).start()
            return carry

        lax.fori_loop(0, u_per_w, unit_body, 0)
        w_copy(u_per_w - 2, 0).wait()
        w_copy(u_per_w - 1, 1).wait()

    return gather_kernel


def kernel(tokens, weight):
    b, s = tokens.shape
    assert b % _LANES == 0
    n_units = s * (b // _LANES)
    tok = tokens.astype(jnp.int32).T.reshape(n_units, _LANES)
    rows = tok >> 1
    half = (tok & 1) * EMBED_DIM
    table = weight.reshape(VOCAB // 2, 2 * EMBED_DIM)
    out = _make_gather(n_units, s, b)(rows, half, table)
    return jnp.transpose(out, (2, 0, 1))
